# baseline (speedup 1.0000x reference)
import jax
import jax.numpy as jnp
from jax.experimental import pallas as pl
from jax.experimental.pallas import tpu as pltpu

_B = 128
_V = 100000
_VB = 4096
_NBLK = (_V + _VB - 1) // _VB


def _argmax_body(x_ref, ids_ref, max_sc, idx_sc):
    i = pl.program_id(0)
    x = x_ref[...]
    col = jax.lax.broadcasted_iota(jnp.int32, x.shape, 1) + i * _VB
    valid = col < _V
    xm = jnp.where(valid, x, -jnp.inf)
    bmax = jnp.max(xm, axis=1, keepdims=True)
    bidx = jnp.min(jnp.where(xm == bmax, col, _V), axis=1, keepdims=True)

    @pl.when(i == 0)
    def _():
        max_sc[...] = bmax
        idx_sc[...] = bidx

    @pl.when(i > 0)
    def _():
        better = bmax > max_sc[...]
        max_sc[...] = jnp.where(better, bmax, max_sc[...])
        idx_sc[...] = jnp.where(better, bidx, idx_sc[...])

    @pl.when(i == _NBLK - 1)
    def _():
        ids_ref[...] = idx_sc[...]


def kernel(logits):
    ids = pl.pallas_call(
        _argmax_body,
        grid=(_NBLK,),
        in_specs=[pl.BlockSpec((_B, _VB), lambda i: (0, i))],
        out_specs=pl.BlockSpec((_B, 1), lambda i: (0, 0)),
        out_shape=jax.ShapeDtypeStruct((_B, 1), jnp.int32),
        scratch_shapes=[
            pltpu.VMEM((_B, 1), jnp.float32),
            pltpu.VMEM((_B, 1), jnp.int32),
        ],
    )(logits)
    return ids.reshape(_B), logits
